# baseline (speedup 1.0000x reference)
import jax
import jax.numpy as jnp
from jax import lax
from jax.experimental import pallas as pl
from jax.experimental.pallas import tpu as pltpu
from jax.experimental.pallas import tpu_sc as plsc

_MAX_H = 64
_MAX_W = 64
_S = _MAX_H * _MAX_W
_B = 4096
_NC, _NS, _L = 2, 16, 16
_NW = _NC * _NS
_RPT = _B // _NW
_CHUNK = 4
_NCH = _RPT // _CHUNK
_NBUF = 3
_GROUPS = _S // _L
_UNROLL = 4


def _sc_body(x_hbm, idx_hbm, pe_hbm, out_hbm,
             pe_raw, pe_s, idx_v, buf_in, buf_out,
             sem_in0, sem_in1, sem_in2, sem_out0, sem_out1, sem_out2):
    sem_in = (sem_in0, sem_in1, sem_in2)
    sem_out = (sem_out0, sem_out1, sem_out2)
    wid = lax.axis_index("s") * _NC + lax.axis_index("c")
    base = wid * _RPT

    pltpu.sync_copy(pe_hbm, pe_raw)
    pltpu.sync_copy(idx_hbm, idx_v)

    def gather_body(g, carry):
        s = pl.ds(g * _L, _L)
        vals = plsc.load_gather(pe_raw, [idx_v[s]])
        pe_s[s] = vals * jnp.float32(0.1)
        return carry

    lax.fori_loop(0, _GROUPS, gather_body, 0)

    def in_cp(c, b):
        return pltpu.make_async_copy(
            x_hbm.at[pl.ds(base + c * _CHUNK, _CHUNK)], buf_in.at[b], sem_in[b])

    def out_cp(c, b):
        return pltpu.make_async_copy(
            buf_out.at[b], out_hbm.at[pl.ds(base + c * _CHUNK, _CHUNK)],
            sem_out[b])

    for b in range(_NBUF):
        in_cp(b, b).start()

    for c in range(_NCH):
        b = c % _NBUF
        in_cp(c, b).wait()
        if c >= _NBUF:
            out_cp(c - _NBUF, b).wait()
        bi = buf_in.at[b]
        bo = buf_out.at[b]

        def add_body(u, carry, bi=bi, bo=bo):
            for k in range(_UNROLL):
                s = pl.ds((u * _UNROLL + k) * _L, _L)
                pe_vec = pe_s[s]
                for r in range(_CHUNK):
                    bo[r, s] = bi[r, s] + pe_vec
            return carry

        lax.fori_loop(0, _GROUPS // _UNROLL, add_body, 0)
        out_cp(c, b).start()
        if c + _NBUF < _NCH:
            in_cp(c + _NBUF, b).start()

    for c in range(_NCH - _NBUF, _NCH):
        out_cp(c, c % _NBUF).wait()


def kernel(x_flat, height, width, pos_embed):
    offset = (jnp.asarray(height, jnp.int32) - _MAX_H) + (
        jnp.asarray(width, jnp.int32) - _MAX_W
    )
    idx = jnp.clip(jnp.arange(_S, dtype=jnp.int32) + offset, 0, _S - 1)
    run = pl.kernel(
        _sc_body,
        out_type=jax.ShapeDtypeStruct((_B, _S), jnp.float32),
        mesh=plsc.VectorSubcoreMesh(core_axis_name="c", subcore_axis_name="s"),
        compiler_params=pltpu.CompilerParams(needs_layout_passes=False),
        scratch_types=[
            pltpu.VMEM((_S,), jnp.float32),
            pltpu.VMEM((_S,), jnp.float32),
            pltpu.VMEM((_S,), jnp.int32),
            pltpu.VMEM((_NBUF, _CHUNK, _S), jnp.float32),
            pltpu.VMEM((_NBUF, _CHUNK, _S), jnp.float32),
            pltpu.SemaphoreType.DMA,
            pltpu.SemaphoreType.DMA,
            pltpu.SemaphoreType.DMA,
            pltpu.SemaphoreType.DMA,
            pltpu.SemaphoreType.DMA,
            pltpu.SemaphoreType.DMA,
        ],
    )
    return run(x_flat, idx, pos_embed)


# speedup vs baseline: 1.9789x; 1.9789x over previous
import jax
import jax.numpy as jnp
from jax import lax
from jax.experimental import pallas as pl
from jax.experimental.pallas import tpu as pltpu
from jax.experimental.pallas import tpu_sc as plsc

_MAX_H = 64
_MAX_W = 64
_S = _MAX_H * _MAX_W
_B = 4096
_NC, _NS, _L = 2, 16, 16
_NW = _NC * _NS
_RPT = _B // _NW
_CHUNK = 4
_NCH = _RPT // _CHUNK
_NBUF = 3
_GROUPS = _S // _L
_UNROLL = 4


def _sc_body(x_hbm, idx_hbm, pe_hbm, out_hbm,
             pe_raw, pe_s, idx_v, buf_in, buf_out,
             sem_in0, sem_in1, sem_in2, sem_out0, sem_out1, sem_out2):
    sem_in = (sem_in0, sem_in1, sem_in2)
    sem_out = (sem_out0, sem_out1, sem_out2)
    wid = lax.axis_index("s") * _NC + lax.axis_index("c")
    base = wid * _RPT

    pltpu.sync_copy(pe_hbm, pe_raw)
    pltpu.sync_copy(idx_hbm, idx_v)

    def gather_body(g, carry):
        s = pl.ds(g * _L, _L)
        vals = plsc.load_gather(pe_raw, [idx_v[s]])
        pe_s[s] = vals * jnp.float32(0.1)
        return carry

    lax.fori_loop(0, _GROUPS, gather_body, 0)

    def in_cp(c, b):
        return pltpu.make_async_copy(
            x_hbm.at[pl.ds(base + c * _CHUNK, _CHUNK)], buf_in.at[b], sem_in[b])

    def out_cp(c, b):
        return pltpu.make_async_copy(
            buf_out.at[b], out_hbm.at[pl.ds(base + c * _CHUNK, _CHUNK)],
            sem_out[b])

    for b in range(_NBUF):
        in_cp(b, b).start()

    for c in range(_NCH):
        b = c % _NBUF
        in_cp(c, b).wait()
        if c >= _NBUF:
            out_cp(c - _NBUF, b).wait()
        bi = buf_in.at[b]
        bo = buf_out.at[b]

        def add_body(g, carry, bi=bi, bo=bo):
            s = pl.ds(g * _L, _L)
            pe_vec = pe_s[s]
            for r in range(_CHUNK):
                bo[r, s] = bi[r, s] + pe_vec
            return carry

        lax.fori_loop(0, _GROUPS, add_body, 0)
        out_cp(c, b).start()
        if c + _NBUF < _NCH:
            in_cp(c + _NBUF, b).start()

    for c in range(_NCH - _NBUF, _NCH):
        out_cp(c, c % _NBUF).wait()


def kernel(x_flat, height, width, pos_embed):
    offset = (jnp.asarray(height, jnp.int32) - _MAX_H) + (
        jnp.asarray(width, jnp.int32) - _MAX_W
    )
    idx = jnp.clip(jnp.arange(_S, dtype=jnp.int32) + offset, 0, _S - 1)
    run = pl.kernel(
        _sc_body,
        out_type=jax.ShapeDtypeStruct((_B, _S), jnp.float32),
        mesh=plsc.VectorSubcoreMesh(core_axis_name="c", subcore_axis_name="s"),
        compiler_params=pltpu.CompilerParams(needs_layout_passes=False),
        scratch_types=[
            pltpu.VMEM((_S,), jnp.float32),
            pltpu.VMEM((_S,), jnp.float32),
            pltpu.VMEM((_S,), jnp.int32),
            pltpu.VMEM((_NBUF, _CHUNK, _S), jnp.float32),
            pltpu.VMEM((_NBUF, _CHUNK, _S), jnp.float32),
            pltpu.SemaphoreType.DMA,
            pltpu.SemaphoreType.DMA,
            pltpu.SemaphoreType.DMA,
            pltpu.SemaphoreType.DMA,
            pltpu.SemaphoreType.DMA,
            pltpu.SemaphoreType.DMA,
        ],
    )
    return run(x_flat, idx, pos_embed)
